# baseline (speedup 1.0000x reference)
import jax
import jax.numpy as jnp
from jax import lax
from jax.experimental import pallas as pl
from jax.experimental.pallas import tpu as pltpu
from jax.experimental.pallas import tpu_sc as plsc

_NC, _NS = 2, 16
_NW = _NC * _NS
_C = 64
_NBUF = 10
_LOOK = 5


def kernel(x, weight):
    B, S = x.shape
    V, D = weight.shape
    n = B * S
    per_tile = n // _NW
    nchunks = per_tile // _C
    ngroups = nchunks // _NBUF
    idx = jnp.swapaxes(x, 0, 1).reshape(n).astype(jnp.int32)

    mesh = plsc.VectorSubcoreMesh(core_axis_name="c", subcore_axis_name="s")

    @pl.kernel(
        out_type=jax.ShapeDtypeStruct((n, D), weight.dtype),
        mesh=mesh,
        scratch_types=[
            pltpu.VMEM((per_tile,), jnp.int32),
            pltpu.VMEM((_NBUF, _C, D), jnp.float32),
        ] + [pltpu.SemaphoreType.DMA] * (2 * _NBUF),
    )
    def k(w_hbm, i_hbm, o_hbm, idx_v, bufs, *sems):
        gsems = sems[:_NBUF]
        wsems = sems[_NBUF:]
        wid = lax.axis_index("s") * _NC + lax.axis_index("c")
        base = wid * per_tile
        pltpu.sync_copy(i_hbm.at[pl.ds(base, per_tile)], idx_v)

        def gather_copy(c, b):
            return pltpu.make_async_copy(
                w_hbm.at[idx_v.at[pl.ds(c * _C, _C)]], bufs.at[b], gsems[b])

        def write_copy(c, b):
            return pltpu.make_async_copy(
                bufs.at[b], o_hbm.at[pl.ds(base + c * _C, _C)], wsems[b])

        for b in range(_NBUF):
            gather_copy(b, b).start()

        for b in range(_NBUF):
            gather_copy(b, b).wait()
            write_copy(b, b).start()
            if b >= _LOOK:
                cp = b - _LOOK
                write_copy(cp, cp).wait()
                gather_copy(cp + _NBUF, cp).start()

        @pl.loop(1, ngroups - 1)
        def _(g):
            for b in range(_NBUF):
                c = g * _NBUF + b
                gather_copy(c, b).wait()
                write_copy(c, b).start()
                bp = (b - _LOOK) % _NBUF
                write_copy(c - _LOOK, bp).wait()
                gather_copy(c - _LOOK + _NBUF, bp).start()

        gl = ngroups - 1
        for b in range(_NBUF):
            c = gl * _NBUF + b
            gather_copy(c, b).wait()
            write_copy(c, b).start()
            if b < _LOOK:
                bp = (b - _LOOK) % _NBUF
                write_copy(c - _LOOK, bp).wait()
                gather_copy(c - _LOOK + _NBUF, bp).start()
        for b in range(_NBUF):
            write_copy(gl * _NBUF + b, b).wait()

    out = k(weight, idx).reshape(S, B, D)
    return jnp.swapaxes(out, 0, 1)


# speedup vs baseline: 1.0007x; 1.0007x over previous
import jax
import jax.numpy as jnp
from jax import lax
from jax.experimental import pallas as pl
from jax.experimental.pallas import tpu as pltpu
from jax.experimental.pallas import tpu_sc as plsc

_NC, _NS = 2, 16
_NW = _NC * _NS
_C = 64
_NBUF = 10
_LOOK = 2


def kernel(x, weight):
    B, S = x.shape
    V, D = weight.shape
    n = B * S
    per_tile = n // _NW
    nchunks = per_tile // _C
    ngroups = nchunks // _NBUF
    idx = jnp.swapaxes(x, 0, 1).reshape(n).astype(jnp.int32)

    mesh = plsc.VectorSubcoreMesh(core_axis_name="c", subcore_axis_name="s")

    @pl.kernel(
        out_type=jax.ShapeDtypeStruct((n, D), weight.dtype),
        mesh=mesh,
        scratch_types=[
            pltpu.VMEM((per_tile,), jnp.int32),
            pltpu.VMEM((_NBUF, _C, D), jnp.float32),
        ] + [pltpu.SemaphoreType.DMA] * (2 * _NBUF),
    )
    def k(w_hbm, i_hbm, o_hbm, idx_v, bufs, *sems):
        gsems = sems[:_NBUF]
        wsems = sems[_NBUF:]
        wid = lax.axis_index("s") * _NC + lax.axis_index("c")
        base = wid * per_tile
        pltpu.sync_copy(i_hbm.at[pl.ds(base, per_tile)], idx_v)

        def gather_copy(c, b):
            return pltpu.make_async_copy(
                w_hbm.at[idx_v.at[pl.ds(c * _C, _C)]], bufs.at[b], gsems[b])

        def write_copy(c, b):
            return pltpu.make_async_copy(
                bufs.at[b], o_hbm.at[pl.ds(base + c * _C, _C)], wsems[b])

        for b in range(_NBUF):
            gather_copy(b, b).start()

        for b in range(_NBUF):
            gather_copy(b, b).wait()
            write_copy(b, b).start()
            if b >= _LOOK:
                cp = b - _LOOK
                write_copy(cp, cp).wait()
                gather_copy(cp + _NBUF, cp).start()

        @pl.loop(1, ngroups - 1)
        def _(g):
            for b in range(_NBUF):
                c = g * _NBUF + b
                gather_copy(c, b).wait()
                write_copy(c, b).start()
                bp = (b - _LOOK) % _NBUF
                write_copy(c - _LOOK, bp).wait()
                gather_copy(c - _LOOK + _NBUF, bp).start()

        gl = ngroups - 1
        for b in range(_NBUF):
            c = gl * _NBUF + b
            gather_copy(c, b).wait()
            write_copy(c, b).start()
            if b < _LOOK:
                bp = (b - _LOOK) % _NBUF
                write_copy(c - _LOOK, bp).wait()
                gather_copy(c - _LOOK + _NBUF, bp).start()
        for b in range(_NBUF):
            write_copy(gl * _NBUF + b, b).wait()

    out = k(weight, idx).reshape(S, B, D)
    return jnp.swapaxes(out, 0, 1)


# speedup vs baseline: 1.0056x; 1.0049x over previous
import jax
import jax.numpy as jnp
from jax import lax
from jax.experimental import pallas as pl
from jax.experimental.pallas import tpu as pltpu
from jax.experimental.pallas import tpu_sc as plsc

_NC, _NS = 2, 16
_NW = _NC * _NS
_C = 80
_NBUF = 10
_LOOK = 3


def kernel(x, weight):
    B, S = x.shape
    V, D = weight.shape
    n = B * S
    per_tile = n // _NW
    nchunks = per_tile // _C
    ngroups = nchunks // _NBUF
    idx = jnp.swapaxes(x, 0, 1).reshape(n).astype(jnp.int32)

    mesh = plsc.VectorSubcoreMesh(core_axis_name="c", subcore_axis_name="s")

    @pl.kernel(
        out_type=jax.ShapeDtypeStruct((n, D), weight.dtype),
        mesh=mesh,
        scratch_types=[
            pltpu.VMEM((per_tile,), jnp.int32),
            pltpu.VMEM((_NBUF, _C, D), jnp.float32),
        ] + [pltpu.SemaphoreType.DMA] * (2 * _NBUF),
    )
    def k(w_hbm, i_hbm, o_hbm, idx_v, bufs, *sems):
        gsems = sems[:_NBUF]
        wsems = sems[_NBUF:]
        wid = lax.axis_index("s") * _NC + lax.axis_index("c")
        base = wid * per_tile
        pltpu.sync_copy(i_hbm.at[pl.ds(base, per_tile)], idx_v)

        def gather_copy(c, b):
            return pltpu.make_async_copy(
                w_hbm.at[idx_v.at[pl.ds(c * _C, _C)]], bufs.at[b], gsems[b])

        def write_copy(c, b):
            return pltpu.make_async_copy(
                bufs.at[b], o_hbm.at[pl.ds(base + c * _C, _C)], wsems[b])

        for b in range(_NBUF):
            gather_copy(b, b).start()

        for b in range(_NBUF):
            gather_copy(b, b).wait()
            write_copy(b, b).start()
            if b >= _LOOK:
                cp = b - _LOOK
                write_copy(cp, cp).wait()
                gather_copy(cp + _NBUF, cp).start()

        @pl.loop(1, ngroups - 1)
        def _(g):
            for b in range(_NBUF):
                c = g * _NBUF + b
                gather_copy(c, b).wait()
                write_copy(c, b).start()
                bp = (b - _LOOK) % _NBUF
                write_copy(c - _LOOK, bp).wait()
                gather_copy(c - _LOOK + _NBUF, bp).start()

        gl = ngroups - 1
        for b in range(_NBUF):
            c = gl * _NBUF + b
            gather_copy(c, b).wait()
            write_copy(c, b).start()
            if b < _LOOK:
                bp = (b - _LOOK) % _NBUF
                write_copy(c - _LOOK, bp).wait()
                gather_copy(c - _LOOK + _NBUF, bp).start()
        for b in range(_NBUF):
            write_copy(gl * _NBUF + b, b).wait()

    out = k(weight, idx).reshape(S, B, D)
    return jnp.swapaxes(out, 0, 1)
